# baseline (speedup 1.0000x reference)
import functools

import jax
import jax.numpy as jnp
from jax import lax
from jax.experimental import pallas as pl
from jax.experimental.pallas import tpu as pltpu
from jax.experimental.pallas import tpu_sc as plsc

NC = 2
NS = 16
BLK = 128
NBUF = 2


def _sc_aggregate(nbr_feat, idx2d, zrows, consts, n_nodes, deg_stripe):
    E, D = nbr_feat.shape
    NW = NC * NS
    nblk_total = E // BLK
    nblk_base = nblk_total // NW
    n_extra = nblk_total - nblk_base * NW
    wid_extra0 = NW - n_extra
    rows_per_tile = n_nodes // NS
    deg_pad = NS * deg_stripe
    ngrp = nblk_base // NBUF
    assert ngrp * NBUF == nblk_base

    mesh = plsc.VectorSubcoreMesh(core_axis_name="c", subcore_axis_name="s")

    @functools.partial(
        pl.kernel,
        mesh=mesh,
        compiler_params=pltpu.CompilerParams(use_tc_tiling_on_sc=False),
        out_type=[
            jax.ShapeDtypeStruct((NC, n_nodes, D), jnp.float32),
            jax.ShapeDtypeStruct((NC, deg_pad), jnp.float32),
        ],
        scratch_types=[
            pltpu.VMEM((nblk_base + 1, BLK), jnp.int32),
            pltpu.VMEM((NBUF, BLK, D), jnp.float32),
            pltpu.VMEM((BLK,), jnp.float32),
            pltpu.VMEM_SHARED((n_nodes, D), jnp.float32),
            pltpu.VMEM_SHARED((deg_pad,), jnp.float32),
            pltpu.SemaphoreType.DMA((NBUF,)),
            pltpu.SemaphoreType.DMA((NBUF,)),
            pltpu.SemaphoreType.DMA,
        ],
    )
    def k(nbr_hbm, idx_hbm, zrows_hbm, consts_hbm, out_sum, out_deg,
          idx_v, buf, ones_v, acc_sh, deg_sh, sem_g, sem_s, sem_d):
        c = lax.axis_index("c")
        s = lax.axis_index("s")
        wid = s * NC + c
        has_extra = wid >= wid_extra0
        blk0 = wid * nblk_base
        extra_blk = nblk_base * NW + (wid - wid_extra0)
        ebase = blk0 * BLK

        for b in range(NBUF):
            pltpu.async_copy(nbr_hbm.at[pl.ds(ebase + b * BLK, BLK)],
                             buf.at[b], sem_g.at[b])

        pltpu.sync_copy(idx_hbm.at[pl.ds(blk0, nblk_base)],
                        idx_v.at[pl.ds(0, nblk_base)])

        @pl.when(has_extra)
        def _():
            pltpu.sync_copy(idx_hbm.at[pl.ds(extra_blk, 1)],
                            idx_v.at[pl.ds(nblk_base, 1)])

        pltpu.sync_copy(consts_hbm.at[pl.ds(deg_stripe, BLK)], ones_v)

        pltpu.sync_copy(zrows_hbm,
                        acc_sh.at[pl.ds(s * rows_per_tile, rows_per_tile)])
        pltpu.sync_copy(consts_hbm.at[pl.ds(0, deg_stripe)],
                        deg_sh.at[pl.ds(s * deg_stripe, deg_stripe)])

        plsc.subcore_barrier()

        def grp_body(g, carry):
            base_blk = g * NBUF
            feat_descs = []
            deg_descs = []
            for b in range(NBUF):
                blk = base_blk + b
                pltpu.make_async_copy(nbr_hbm.at[pl.ds(ebase, BLK)],
                                      buf.at[b], sem_g.at[b]).wait()
                feat_descs.append(pltpu.async_copy(
                    buf.at[b], acc_sh.at[idx_v.at[blk]], sem_s.at[b],
                    add=True))
                deg_descs.append(pltpu.async_copy(
                    ones_v, deg_sh.at[idx_v.at[blk]], sem_d, add=True))
            for b in range(NBUF):
                feat_descs[b].wait()
                deg_descs[b].wait()
                blk_next = base_blk + NBUF + b

                @pl.when(blk_next < nblk_base)
                def _():
                    pltpu.async_copy(
                        nbr_hbm.at[pl.ds(ebase + blk_next * BLK, BLK)],
                        buf.at[b], sem_g.at[b])
            return carry

        lax.fori_loop(0, ngrp, grp_body, None)

        @pl.when(has_extra)
        def _():
            pltpu.sync_copy(nbr_hbm.at[pl.ds(extra_blk * BLK, BLK)],
                            buf.at[0])
            pltpu.sync_copy(buf.at[0], acc_sh.at[idx_v.at[nblk_base]],
                            add=True)
            pltpu.sync_copy(ones_v, deg_sh.at[idx_v.at[nblk_base]], add=True)

        plsc.subcore_barrier()

        pltpu.sync_copy(acc_sh.at[pl.ds(s * rows_per_tile, rows_per_tile)],
                        out_sum.at[c, pl.ds(s * rows_per_tile, rows_per_tile)])
        pltpu.sync_copy(deg_sh.at[pl.ds(s * deg_stripe, deg_stripe)],
                        out_deg.at[c, pl.ds(s * deg_stripe, deg_stripe)])

    return k(nbr_feat, idx2d, zrows, consts)


def _tc_finish(self_feat, psum, pdeg, W):
    N, D = self_feat.shape

    def body(self_ref, p_ref, d_ref, w_ref, o_ref):
        x = self_ref[...] + p_ref[0] + p_ref[1]
        deg = d_ref[0] + d_ref[1] + 1.0
        y = x / deg
        o_ref[...] = lax.dot_general(
            y, w_ref[...],
            dimension_numbers=(((1,), (1,)), ((), ())),
            preferred_element_type=jnp.float32)

    return pl.pallas_call(
        body,
        out_shape=jax.ShapeDtypeStruct((N, D), jnp.float32),
    )(self_feat, psum, pdeg, W)


def kernel(self_feat, nbr_feat, relation_src_indices, W):
    N, D = self_feat.shape
    E = nbr_feat.shape[0]
    assert E % BLK == 0 and N % NS == 0
    deg_stripe = -(-(N // NS) // 128) * 128

    idx2d = relation_src_indices.astype(jnp.int32).reshape(E // BLK, BLK)
    zrows = jnp.zeros((N // NS, D), jnp.float32)
    consts = jnp.concatenate(
        [jnp.zeros((deg_stripe,), jnp.float32),
         jnp.ones((BLK,), jnp.float32)])
    psum, pdeg = _sc_aggregate(nbr_feat, idx2d, zrows, consts, N, deg_stripe)
    pdeg3 = pdeg[:, :N].reshape(NC, N, 1)
    return _tc_finish(self_feat, psum, pdeg3, W)


# speedup vs baseline: 1.0537x; 1.0537x over previous
import functools

import jax
import jax.numpy as jnp
from jax import lax
from jax.experimental import pallas as pl
from jax.experimental.pallas import tpu as pltpu
from jax.experimental.pallas import tpu_sc as plsc

NC = 2
NS = 16
BLK = 128
NBUF = 2


def _sc_aggregate(nbr_feat, idx2d, zrows, consts, n_nodes, deg_stripe):
    E, D = nbr_feat.shape
    NW = NC * NS
    nblk_total = E // BLK
    nblk_base = nblk_total // NW
    n_extra = nblk_total - nblk_base * NW
    wid_extra0 = NW - n_extra
    rows_per_tile = n_nodes // NS
    deg_pad = NS * deg_stripe
    ngrp = nblk_base // NBUF
    assert ngrp * NBUF == nblk_base

    mesh = plsc.VectorSubcoreMesh(core_axis_name="c", subcore_axis_name="s")

    @functools.partial(
        pl.kernel,
        mesh=mesh,
        compiler_params=pltpu.CompilerParams(use_tc_tiling_on_sc=False),
        out_type=[
            jax.ShapeDtypeStruct((NC, n_nodes, D), jnp.float32),
            jax.ShapeDtypeStruct((NC, deg_pad), jnp.float32),
        ],
        scratch_types=[
            pltpu.VMEM((nblk_base + 1, BLK), jnp.int32),
            pltpu.VMEM((NBUF, BLK, D), jnp.float32),
            pltpu.VMEM((BLK,), jnp.float32),
            pltpu.VMEM_SHARED((n_nodes, D), jnp.float32),
            pltpu.VMEM_SHARED((deg_pad,), jnp.float32),
            pltpu.SemaphoreType.DMA((NBUF,)),
            pltpu.SemaphoreType.DMA((NBUF,)),
            pltpu.SemaphoreType.DMA,
        ],
    )
    def k(nbr_hbm, idx_hbm, zrows_hbm, consts_hbm, out_sum, out_deg,
          idx_v, buf, ones_v, acc_sh, deg_sh, sem_g, sem_s, sem_d):
        c = lax.axis_index("c")
        s = lax.axis_index("s")
        wid = s * NC + c
        has_extra = wid >= wid_extra0
        blk0 = wid * nblk_base
        extra_blk = nblk_base * NW + (wid - wid_extra0)
        ebase = blk0 * BLK

        for b in range(NBUF):
            pltpu.async_copy(nbr_hbm.at[pl.ds(ebase + b * BLK, BLK)],
                             buf.at[b], sem_g.at[b])

        pltpu.sync_copy(idx_hbm.at[pl.ds(blk0, nblk_base)],
                        idx_v.at[pl.ds(0, nblk_base)])

        @pl.when(has_extra)
        def _():
            pltpu.sync_copy(idx_hbm.at[pl.ds(extra_blk, 1)],
                            idx_v.at[pl.ds(nblk_base, 1)])

        pltpu.sync_copy(consts_hbm.at[pl.ds(deg_stripe, BLK)], ones_v)

        pltpu.sync_copy(zrows_hbm,
                        acc_sh.at[pl.ds(s * rows_per_tile, rows_per_tile)])
        pltpu.sync_copy(consts_hbm.at[pl.ds(0, deg_stripe)],
                        deg_sh.at[pl.ds(s * deg_stripe, deg_stripe)])

        plsc.subcore_barrier()

        def grp_body(g, carry):
            base_blk = g * NBUF
            feat_descs = []
            deg_descs = []
            for b in range(NBUF):
                blk = base_blk + b
                pltpu.make_async_copy(nbr_hbm.at[pl.ds(ebase, BLK)],
                                      buf.at[b], sem_g.at[b]).wait()
                feat_descs.append(pltpu.async_copy(
                    buf.at[b], acc_sh.at[idx_v.at[blk]], sem_s.at[b],
                    add=True))
                deg_descs.append(pltpu.async_copy(
                    ones_v, deg_sh.at[idx_v.at[blk]], sem_d, add=True))
            for b in range(NBUF):
                feat_descs[b].wait()
                deg_descs[b].wait()
                blk_next = base_blk + NBUF + b

                @pl.when(blk_next < nblk_base)
                def _():
                    pltpu.async_copy(
                        nbr_hbm.at[pl.ds(ebase + blk_next * BLK, BLK)],
                        buf.at[b], sem_g.at[b])
            return carry

        lax.fori_loop(0, ngrp, grp_body, None)

        @pl.when(has_extra)
        def _():
            pltpu.sync_copy(nbr_hbm.at[pl.ds(extra_blk * BLK, BLK)],
                            buf.at[0])
            pltpu.sync_copy(buf.at[0], acc_sh.at[idx_v.at[nblk_base]],
                            add=True)
            pltpu.sync_copy(ones_v, deg_sh.at[idx_v.at[nblk_base]], add=True)

        plsc.subcore_barrier()

        pltpu.sync_copy(acc_sh.at[pl.ds(s * rows_per_tile, rows_per_tile)],
                        out_sum.at[c, pl.ds(s * rows_per_tile, rows_per_tile)])
        pltpu.sync_copy(deg_sh.at[pl.ds(s * deg_stripe, deg_stripe)],
                        out_deg.at[c, pl.ds(s * deg_stripe, deg_stripe)])

    return k(nbr_feat, idx2d, zrows, consts)


def _tc_finish(self_feat, psum, pdeg2, W):
    N, D = self_feat.shape
    nfull, tail_rows = divmod(N, 128)

    def body(self_ref, p_ref, d_ref, w_ref, o_ref):
        x = self_ref[...] + p_ref[0] + p_ref[1]
        r2 = 1.0 / (d_ref[0] + d_ref[1] + 1.0)
        r2t = r2.T
        parts = []
        for a in range(nfull):
            col = lax.slice(r2t, (0, a), (128, a + 1))
            parts.append(x[a * 128:(a + 1) * 128, :] * col)
        if tail_rows:
            col = lax.slice(r2t, (0, nfull), (tail_rows, nfull + 1))
            parts.append(x[nfull * 128:N, :] * col)
        y = jnp.concatenate(parts, axis=0)
        o_ref[...] = lax.dot_general(
            y, w_ref[...],
            dimension_numbers=(((1,), (1,)), ((), ())),
            preferred_element_type=jnp.float32)

    return pl.pallas_call(
        body,
        out_shape=jax.ShapeDtypeStruct((N, D), jnp.float32),
    )(self_feat, psum, pdeg2, W)


def kernel(self_feat, nbr_feat, relation_src_indices, W):
    N, D = self_feat.shape
    E = nbr_feat.shape[0]
    assert E % BLK == 0 and N % NS == 0
    deg_stripe = -(-(N // NS) // 128) * 128

    idx2d = relation_src_indices.astype(jnp.int32).reshape(E // BLK, BLK)
    zrows = jnp.zeros((N // NS, D), jnp.float32)
    consts = jnp.concatenate(
        [jnp.zeros((deg_stripe,), jnp.float32),
         jnp.ones((BLK,), jnp.float32)])
    psum, pdeg = _sc_aggregate(nbr_feat, idx2d, zrows, consts, N, deg_stripe)
    pdeg2 = pdeg.reshape(NC, -1, 128)
    return _tc_finish(self_feat, psum, pdeg2, W)


# speedup vs baseline: 1.1273x; 1.0699x over previous
import functools

import jax
import jax.numpy as jnp
from jax import lax
from jax.experimental import pallas as pl
from jax.experimental.pallas import tpu as pltpu
from jax.experimental.pallas import tpu_sc as plsc

NC = 2
NS = 16
BLK = 128
NBUF = 3


def _sc_aggregate(nbr_feat, idx2d, zrows, consts, n_nodes, deg_stripe):
    E, D = nbr_feat.shape
    NW = NC * NS
    nblk_total = E // BLK
    nblk_base = nblk_total // NW
    n_extra = nblk_total - nblk_base * NW
    wid_extra0 = NW - n_extra
    rows_per_tile = n_nodes // NS
    deg_pad = NS * deg_stripe
    ngrp = nblk_base // NBUF
    assert ngrp * NBUF == nblk_base

    mesh = plsc.VectorSubcoreMesh(core_axis_name="c", subcore_axis_name="s")

    @functools.partial(
        pl.kernel,
        mesh=mesh,
        compiler_params=pltpu.CompilerParams(use_tc_tiling_on_sc=False),
        out_type=[
            jax.ShapeDtypeStruct((NC, n_nodes, D), jnp.float32),
            jax.ShapeDtypeStruct((NC, deg_pad), jnp.float32),
        ],
        scratch_types=[
            pltpu.VMEM((NBUF, 1, BLK), jnp.int32),
            pltpu.VMEM((NBUF, BLK, D), jnp.float32),
            pltpu.VMEM((BLK,), jnp.float32),
            pltpu.VMEM_SHARED((n_nodes, D), jnp.float32),
            pltpu.VMEM_SHARED((deg_pad,), jnp.float32),
            pltpu.SemaphoreType.DMA((NBUF,)),
            pltpu.SemaphoreType.DMA((NBUF,)),
            pltpu.SemaphoreType.DMA((NBUF,)),
            pltpu.SemaphoreType.DMA,
        ],
    )
    def k(nbr_hbm, idx_hbm, zrows_hbm, consts_hbm, out_sum, out_deg,
          idx_r, buf, ones_v, acc_sh, deg_sh, sem_g, sem_i, sem_s, sem_d):
        c = lax.axis_index("c")
        s = lax.axis_index("s")
        wid = s * NC + c
        has_extra = wid >= wid_extra0
        blk0 = wid * nblk_base
        extra_blk = nblk_base * NW + (wid - wid_extra0)
        ebase = blk0 * BLK

        for b in range(NBUF):
            pltpu.async_copy(nbr_hbm.at[pl.ds(ebase + b * BLK, BLK)],
                             buf.at[b], sem_g.at[b])
            pltpu.async_copy(idx_hbm.at[pl.ds(blk0 + b, 1)],
                             idx_r.at[b], sem_i.at[b])

        pltpu.sync_copy(consts_hbm.at[pl.ds(deg_stripe, BLK)], ones_v)

        pltpu.sync_copy(zrows_hbm,
                        acc_sh.at[pl.ds(s * rows_per_tile, rows_per_tile)])
        pltpu.sync_copy(consts_hbm.at[pl.ds(0, deg_stripe)],
                        deg_sh.at[pl.ds(s * deg_stripe, deg_stripe)])

        plsc.subcore_barrier()

        def grp_body(g, carry):
            base_blk = g * NBUF
            feat_descs = []
            deg_descs = []
            for b in range(NBUF):
                pltpu.make_async_copy(nbr_hbm.at[pl.ds(ebase, BLK)],
                                      buf.at[b], sem_g.at[b]).wait()
                pltpu.make_async_copy(idx_hbm.at[pl.ds(blk0, 1)],
                                      idx_r.at[b], sem_i.at[b]).wait()
                feat_descs.append(pltpu.async_copy(
                    buf.at[b], acc_sh.at[idx_r.at[b, 0]], sem_s.at[b],
                    add=True))
                deg_descs.append(pltpu.async_copy(
                    ones_v, deg_sh.at[idx_r.at[b, 0]], sem_d, add=True))
            for b in range(NBUF):
                feat_descs[b].wait()
                deg_descs[b].wait()
                blk_next = base_blk + NBUF + b

                @pl.when(blk_next < nblk_base)
                def _():
                    pltpu.async_copy(
                        nbr_hbm.at[pl.ds(ebase + blk_next * BLK, BLK)],
                        buf.at[b], sem_g.at[b])
                    pltpu.async_copy(idx_hbm.at[pl.ds(blk0 + blk_next, 1)],
                                     idx_r.at[b], sem_i.at[b])
            return carry

        lax.fori_loop(0, ngrp, grp_body, None)

        @pl.when(has_extra)
        def _():
            pltpu.sync_copy(idx_hbm.at[pl.ds(extra_blk, 1)], idx_r.at[0])
            pltpu.sync_copy(nbr_hbm.at[pl.ds(extra_blk * BLK, BLK)],
                            buf.at[0])
            pltpu.sync_copy(buf.at[0], acc_sh.at[idx_r.at[0, 0]], add=True)
            pltpu.sync_copy(ones_v, deg_sh.at[idx_r.at[0, 0]], add=True)

        plsc.subcore_barrier()

        pltpu.sync_copy(acc_sh.at[pl.ds(s * rows_per_tile, rows_per_tile)],
                        out_sum.at[c, pl.ds(s * rows_per_tile, rows_per_tile)])
        pltpu.sync_copy(deg_sh.at[pl.ds(s * deg_stripe, deg_stripe)],
                        out_deg.at[c, pl.ds(s * deg_stripe, deg_stripe)])

    return k(nbr_feat, idx2d, zrows, consts)


def _tc_finish(self_feat, psum, pdeg2, W):
    N, D = self_feat.shape
    nfull, tail_rows = divmod(N, 128)

    def body(self_ref, p_ref, d_ref, w_ref, o_ref):
        x = self_ref[...] + p_ref[0] + p_ref[1]
        r2 = 1.0 / (d_ref[0] + d_ref[1] + 1.0)
        r2t = r2.T
        parts = []
        for a in range(nfull):
            col = lax.slice(r2t, (0, a), (128, a + 1))
            parts.append(x[a * 128:(a + 1) * 128, :] * col)
        if tail_rows:
            col = lax.slice(r2t, (0, nfull), (tail_rows, nfull + 1))
            parts.append(x[nfull * 128:N, :] * col)
        y = jnp.concatenate(parts, axis=0)
        o_ref[...] = lax.dot_general(
            y, w_ref[...],
            dimension_numbers=(((1,), (1,)), ((), ())),
            preferred_element_type=jnp.float32)

    return pl.pallas_call(
        body,
        out_shape=jax.ShapeDtypeStruct((N, D), jnp.float32),
    )(self_feat, psum, pdeg2, W)


def kernel(self_feat, nbr_feat, relation_src_indices, W):
    N, D = self_feat.shape
    E = nbr_feat.shape[0]
    assert E % BLK == 0 and N % NS == 0
    deg_stripe = -(-(N // NS) // 128) * 128

    idx2d = relation_src_indices.astype(jnp.int32).reshape(E // BLK, BLK)
    zrows = jnp.zeros((N // NS, D), jnp.float32)
    consts = jnp.concatenate(
        [jnp.zeros((deg_stripe,), jnp.float32),
         jnp.ones((BLK,), jnp.float32)])
    psum, pdeg = _sc_aggregate(nbr_feat, idx2d, zrows, consts, N, deg_stripe)
    pdeg2 = pdeg.reshape(NC, -1, 128)
    return _tc_finish(self_feat, psum, pdeg2, W)


# speedup vs baseline: 1.1376x; 1.0091x over previous
import functools

import jax
import jax.numpy as jnp
from jax import lax
from jax.experimental import pallas as pl
from jax.experimental.pallas import tpu as pltpu
from jax.experimental.pallas import tpu_sc as plsc

NC = 2
NS = 16
BLK = 128
NBUF = 3


def _sc_aggregate(nbr_feat, idx2d, self_feat, zrows, consts, n_nodes,
                  deg_stripe):
    E, D = nbr_feat.shape
    NW = NC * NS
    nblk_total = E // BLK
    nblk_base = nblk_total // NW
    n_extra = nblk_total - nblk_base * NW
    wid_extra0 = NW - n_extra
    rows_per_tile = n_nodes // NS
    deg_pad = NS * deg_stripe
    ngrp = nblk_base // NBUF
    assert ngrp * NBUF == nblk_base

    mesh = plsc.VectorSubcoreMesh(core_axis_name="c", subcore_axis_name="s")

    @functools.partial(
        pl.kernel,
        mesh=mesh,
        compiler_params=pltpu.CompilerParams(use_tc_tiling_on_sc=False),
        out_type=[
            jax.ShapeDtypeStruct((NC, n_nodes, D), jnp.float32),
            jax.ShapeDtypeStruct((NC, deg_pad), jnp.float32),
        ],
        scratch_types=[
            pltpu.VMEM((NBUF, 1, BLK), jnp.int32),
            pltpu.VMEM((NBUF, BLK, D), jnp.float32),
            pltpu.VMEM((BLK,), jnp.float32),
            pltpu.VMEM_SHARED((n_nodes, D), jnp.float32),
            pltpu.VMEM_SHARED((deg_pad,), jnp.float32),
            pltpu.SemaphoreType.DMA((NBUF,)),
            pltpu.SemaphoreType.DMA((NBUF,)),
            pltpu.SemaphoreType.DMA((NBUF,)),
            pltpu.SemaphoreType.DMA,
        ],
    )
    def k(nbr_hbm, idx_hbm, self_hbm, zrows_hbm, consts_hbm, out_sum, out_deg,
          idx_r, buf, ones_v, acc_sh, deg_sh, sem_g, sem_i, sem_s, sem_d):
        c = lax.axis_index("c")
        s = lax.axis_index("s")
        wid = s * NC + c
        has_extra = wid >= wid_extra0
        blk0 = wid * nblk_base
        extra_blk = nblk_base * NW + (wid - wid_extra0)
        ebase = blk0 * BLK

        for b in range(NBUF):
            pltpu.async_copy(nbr_hbm.at[pl.ds(ebase + b * BLK, BLK)],
                             buf.at[b], sem_g.at[b])
            pltpu.async_copy(idx_hbm.at[pl.ds(blk0 + b, 1)],
                             idx_r.at[b], sem_i.at[b])

        pltpu.sync_copy(consts_hbm.at[pl.ds(deg_stripe, BLK)], ones_v)

        @pl.when(c == 0)
        def _():
            pltpu.sync_copy(
                self_hbm.at[pl.ds(s * rows_per_tile, rows_per_tile)],
                acc_sh.at[pl.ds(s * rows_per_tile, rows_per_tile)])

        @pl.when(c == 1)
        def _():
            pltpu.sync_copy(
                zrows_hbm,
                acc_sh.at[pl.ds(s * rows_per_tile, rows_per_tile)])
        pltpu.sync_copy(consts_hbm.at[pl.ds(0, deg_stripe)],
                        deg_sh.at[pl.ds(s * deg_stripe, deg_stripe)])

        plsc.subcore_barrier()

        def grp_body(g, carry):
            base_blk = g * NBUF
            feat_descs = []
            deg_descs = []
            for b in range(NBUF):
                pltpu.make_async_copy(nbr_hbm.at[pl.ds(ebase, BLK)],
                                      buf.at[b], sem_g.at[b]).wait()
                pltpu.make_async_copy(idx_hbm.at[pl.ds(blk0, 1)],
                                      idx_r.at[b], sem_i.at[b]).wait()
                feat_descs.append(pltpu.async_copy(
                    buf.at[b], acc_sh.at[idx_r.at[b, 0]], sem_s.at[b],
                    add=True))
                deg_descs.append(pltpu.async_copy(
                    ones_v, deg_sh.at[idx_r.at[b, 0]], sem_d, add=True))
            for b in range(NBUF):
                feat_descs[b].wait()
                deg_descs[b].wait()
                blk_next = base_blk + NBUF + b

                @pl.when(blk_next < nblk_base)
                def _():
                    pltpu.async_copy(
                        nbr_hbm.at[pl.ds(ebase + blk_next * BLK, BLK)],
                        buf.at[b], sem_g.at[b])
                    pltpu.async_copy(idx_hbm.at[pl.ds(blk0 + blk_next, 1)],
                                     idx_r.at[b], sem_i.at[b])
            return carry

        lax.fori_loop(0, ngrp, grp_body, None)

        @pl.when(has_extra)
        def _():
            pltpu.sync_copy(idx_hbm.at[pl.ds(extra_blk, 1)], idx_r.at[0])
            pltpu.sync_copy(nbr_hbm.at[pl.ds(extra_blk * BLK, BLK)],
                            buf.at[0])
            pltpu.sync_copy(buf.at[0], acc_sh.at[idx_r.at[0, 0]], add=True)
            pltpu.sync_copy(ones_v, deg_sh.at[idx_r.at[0, 0]], add=True)

        plsc.subcore_barrier()

        pltpu.sync_copy(acc_sh.at[pl.ds(s * rows_per_tile, rows_per_tile)],
                        out_sum.at[c, pl.ds(s * rows_per_tile, rows_per_tile)])
        pltpu.sync_copy(deg_sh.at[pl.ds(s * deg_stripe, deg_stripe)],
                        out_deg.at[c, pl.ds(s * deg_stripe, deg_stripe)])

    return k(nbr_feat, idx2d, self_feat, zrows, consts)


def _tc_finish(n_nodes, psum, pdeg2, W):
    N = n_nodes
    D = psum.shape[-1]
    nfull, tail_rows = divmod(N, 128)

    def body(p_ref, d_ref, w_ref, o_ref):
        x = p_ref[0] + p_ref[1]
        r2 = 1.0 / (d_ref[0] + d_ref[1] + 1.0)
        r2t = r2.T
        parts = []
        for a in range(nfull):
            col = lax.slice(r2t, (0, a), (128, a + 1))
            parts.append(x[a * 128:(a + 1) * 128, :] * col)
        if tail_rows:
            col = lax.slice(r2t, (0, nfull), (tail_rows, nfull + 1))
            parts.append(x[nfull * 128:N, :] * col)
        y = jnp.concatenate(parts, axis=0)
        o_ref[...] = lax.dot_general(
            y, w_ref[...],
            dimension_numbers=(((1,), (1,)), ((), ())),
            preferred_element_type=jnp.float32)

    return pl.pallas_call(
        body,
        out_shape=jax.ShapeDtypeStruct((N, D), jnp.float32),
    )(psum, pdeg2, W)


def kernel(self_feat, nbr_feat, relation_src_indices, W):
    N, D = self_feat.shape
    E = nbr_feat.shape[0]
    assert E % BLK == 0 and N % NS == 0
    deg_stripe = -(-(N // NS) // 128) * 128

    idx2d = relation_src_indices.astype(jnp.int32).reshape(E // BLK, BLK)
    zrows = jnp.zeros((N // NS, D), jnp.float32)
    consts = jnp.concatenate(
        [jnp.zeros((deg_stripe,), jnp.float32),
         jnp.ones((BLK,), jnp.float32)])
    psum, pdeg = _sc_aggregate(nbr_feat, idx2d, self_feat, zrows, consts, N,
                               deg_stripe)
    pdeg2 = pdeg.reshape(NC, -1, 128)
    return _tc_finish(N, psum, pdeg2, W)
